# baseline (speedup 1.0000x reference)
import functools

import jax
import jax.numpy as jnp
from jax import lax
from jax.experimental import pallas as pl
from jax.experimental.pallas import tpu as pltpu
from jax.experimental.pallas import tpu_sc as plsc

_NC = 2
_NS = 16
_NW = _NC * _NS
_L = 16

_SUB = 80
_NSUB = 5
_CHUNK = _SUB * _NSUB


@functools.partial(jax.jit, static_argnums=(3, 4))
def _emb(x_flat, token_table, pos_flat, seq, d):
    n_rows = x_flat.shape[0]
    n_w = n_rows // _NW
    n_chunks = n_w // _CHUNK
    reps = _CHUNK // seq
    dvr = d // _L

    mesh = plsc.VectorSubcoreMesh(core_axis_name="c", subcore_axis_name="s")

    @functools.partial(
        pl.kernel,
        out_type=jax.ShapeDtypeStruct((n_rows, d), jnp.float32),
        mesh=mesh,
        compiler_params=pltpu.CompilerParams(use_tc_tiling_on_sc=False),
        scratch_types=[
            pltpu.VMEM((_CHUNK,), jnp.int32),
            pltpu.VMEM((_CHUNK, d), jnp.float32),
            pltpu.VMEM((seq * d,), jnp.float32),
            pltpu.SemaphoreType.DMA,
        ],
    )
    def body(x_hbm, tab_hbm, pos_hbm, out_hbm, idx_v, rows_v, pos_v, sem):
        wid = lax.axis_index("s") * _NC + lax.axis_index("c")
        base = wid * n_w
        pltpu.sync_copy(pos_hbm, pos_v)

        def do_chunk(i, carry):
            off = base + i * _CHUNK
            pltpu.sync_copy(x_hbm.at[pl.ds(off, _CHUNK)], idx_v)
            cps = [
                pltpu.async_copy(
                    tab_hbm.at[idx_v.at[pl.ds(j * _SUB, _SUB)]],
                    rows_v.at[pl.ds(j * _SUB, _SUB)],
                    sem,
                )
                for j in range(_NSUB)
            ]
            for cp in cps:
                cp.wait()

            def s_loop(s, c2):
                for dd in range(dvr):
                    pv = pos_v[pl.ds(s * d + dd * _L, _L)]
                    for q in range(reps):
                        r = q * seq + s
                        rows_v[r, pl.ds(dd * _L, _L)] += pv
                return c2

            lax.fori_loop(0, seq, s_loop, 0)
            pltpu.sync_copy(rows_v, out_hbm.at[pl.ds(off, _CHUNK)])
            return carry

        lax.fori_loop(0, n_chunks, do_chunk, 0)

    return body(x_flat, token_table, pos_flat)


def kernel(x, token_table, pos_table):
    b, s = x.shape
    v, d = token_table.shape
    n = b * s
    x_flat = x.reshape(n).astype(jnp.int32)
    pos_flat = pos_table[:s].reshape(s * d)
    out = _emb(x_flat, token_table, pos_flat, s, d)
    return out.reshape(b, s, d)


# speedup vs baseline: 1.0812x; 1.0812x over previous
import functools

import jax
import jax.numpy as jnp
from jax import lax
from jax.experimental import pallas as pl
from jax.experimental.pallas import tpu as pltpu
from jax.experimental.pallas import tpu_sc as plsc

_NC = 2
_NS = 16
_NW = _NC * _NS
_L = 16

_SUB = 80
_NSUB = 5
_CHUNK = _SUB * _NSUB


@functools.partial(jax.jit, static_argnums=(3, 4))
def _emb(x_flat, token_table, pos_flat, seq, d):
    n_rows = x_flat.shape[0]
    n_w = n_rows // _NW
    n_chunks = n_w // _CHUNK
    n2 = n_chunks // 2
    reps = _CHUNK // seq
    dvr = d // _L

    mesh = plsc.VectorSubcoreMesh(core_axis_name="c", subcore_axis_name="s")

    @functools.partial(
        pl.kernel,
        out_type=jax.ShapeDtypeStruct((n_rows, d), jnp.float32),
        mesh=mesh,
        compiler_params=pltpu.CompilerParams(use_tc_tiling_on_sc=False),
        scratch_types=[
            pltpu.VMEM((_CHUNK,), jnp.int32),
            pltpu.VMEM((_CHUNK,), jnp.int32),
            pltpu.VMEM((_CHUNK, d), jnp.float32),
            pltpu.VMEM((_CHUNK, d), jnp.float32),
            pltpu.VMEM((seq * d,), jnp.float32),
            pltpu.SemaphoreType.DMA,
            pltpu.SemaphoreType.DMA,
            pltpu.SemaphoreType.DMA,
            pltpu.SemaphoreType.DMA,
        ],
    )
    def body(x_hbm, tab_hbm, pos_hbm, out_hbm, idx0, idx1, rows0, rows1,
             pos_v, gsem0, gsem1, ssem0, ssem1):
        idxs = (idx0, idx1)
        rows = (rows0, rows1)
        gsems = (gsem0, gsem1)
        ssems = (ssem0, ssem1)

        wid = lax.axis_index("s") * _NC + lax.axis_index("c")
        base = wid * n_w
        pltpu.sync_copy(pos_hbm, pos_v)

        def fire_gather(chunk_i, b):
            off = base + chunk_i * _CHUNK
            pltpu.sync_copy(x_hbm.at[pl.ds(off, _CHUNK)], idxs[b])
            for j in range(_NSUB):
                pltpu.async_copy(
                    tab_hbm.at[idxs[b].at[pl.ds(j * _SUB, _SUB)]],
                    rows[b].at[pl.ds(j * _SUB, _SUB)],
                    gsems[b],
                )

        def drain_gather(b):
            pltpu.make_async_copy(
                out_hbm.at[pl.ds(0, _CHUNK)], rows[b], gsems[b]
            ).wait()

        def fire_store(chunk_i, b):
            off = base + chunk_i * _CHUNK
            pltpu.async_copy(rows[b], out_hbm.at[pl.ds(off, _CHUNK)], ssems[b])

        def wait_store(b):
            pltpu.make_async_copy(
                rows[b], out_hbm.at[pl.ds(0, _CHUNK)], ssems[b]
            ).wait()

        def add_pos(b):
            @plsc.parallel_loop(0, seq, unroll=4)
            def _(s):
                for dd in range(dvr):
                    pv = pos_v[pl.ds(s * d + dd * _L, _L)]
                    for q in range(reps):
                        plsc.addupdate(
                            rows[b].at[q * seq + s, pl.ds(dd * _L, _L)], pv
                        )

        fire_gather(0, 0)

        def jbody(j, carry):
            @pl.when(j > 0)
            def _():
                wait_store(1)

            fire_gather(2 * j + 1, 1)
            drain_gather(0)
            add_pos(0)
            fire_store(2 * j, 0)

            @pl.when(j < n2 - 1)
            def _():
                wait_store(0)
                fire_gather(2 * j + 2, 0)

            drain_gather(1)
            add_pos(1)
            fire_store(2 * j + 1, 1)
            return carry

        lax.fori_loop(0, n2, jbody, 0)
        wait_store(0)
        wait_store(1)

    return body(x_flat, token_table, pos_flat)


def kernel(x, token_table, pos_table):
    b, s = x.shape
    v, d = token_table.shape
    n = b * s
    x_flat = x.reshape(n).astype(jnp.int32)
    pos_flat = pos_table[:s].reshape(s * d)
    out = _emb(x_flat, token_table, pos_flat, s, d)
    return out.reshape(b, s, d)
